# Initial kernel scaffold; baseline (speedup 1.0000x reference)
import jax
import jax.numpy as jnp
from jax.experimental import pallas as pl


def kernel(x_nchw, w1, bn1_s, bn1_b, wd1, bnd1_s, bnd1_b, wd2, bnd2_s, bnd2_b, w_se1, w_se2, w2, bn2_s, bn2_b):
    raise NotImplementedError("write your pallas kernel here")


# speedup vs baseline: 1.2640x; 1.2640x over previous
import functools

import jax
import jax.numpy as jnp
from jax import lax
from jax.experimental import pallas as pl
from jax.experimental.pallas import tpu as pltpu


def _hswish(v):
    return v * jnp.clip(v + 3.0, 0.0, 6.0) * (1.0 / 6.0)


def _block_kernel(x_ref, w1_ref, b1_ref, wd1_ref, bd1_ref, wd2_ref, bd2_ref,
                  wse1a_ref, wse1b_ref, wse2a_ref, wse2b_ref,
                  w2a_ref, w2b_ref, b2_ref, o_ref):
    _, inC, H, W = x_ref.shape
    exp = w1_ref.shape[1]
    oup = o_ref.shape[1]
    Mo = H * W
    f32 = jnp.float32

    xs = x_ref[0].reshape(inC, Mo).astype(jnp.bfloat16)
    y = lax.dot_general(xs, w1_ref[...], (((0,), (0,)), ((), ())),
                        preferred_element_type=f32)
    y = _hswish(y + b1_ref[...])
    y3 = y.reshape(H, W, exp)

    wd1 = wd1_ref[...]
    wd2 = wd2_ref[...]
    bd1 = bd1_ref[...]
    bd2 = bd2_ref[...]

    S = jnp.sum(y, axis=0, keepdims=True)
    cs0 = jnp.sum(y3[:, 0, :], axis=0, keepdims=True)
    csW = jnp.sum(y3[:, W - 1, :], axis=0, keepdims=True)
    rs0 = jnp.sum(y3[0], axis=0, keepdims=True)
    rsH = jnp.sum(y3[H - 1], axis=0, keepdims=True)
    inv = 1.0 / float(Mo)
    p1 = (wd1[0:1] * (S - csW) + wd1[1:2] * S + wd1[2:3] * (S - cs0)) * inv + bd1
    p2 = (wd2[0:1] * (S - rsH) + wd2[1:2] * S + wd2[2:3] * (S - rs0)) * inv + bd2

    h = (jnp.dot(p1, wse1a_ref[...], preferred_element_type=f32)
         + jnp.dot(p2, wse1b_ref[...], preferred_element_type=f32))
    h = jnp.maximum(h, 0.0)
    se1 = jnp.clip(jnp.dot(h, wse2a_ref[...], preferred_element_type=f32)
                   + 3.0, 0.0, 6.0) * (1.0 / 6.0)
    se2 = jnp.clip(jnp.dot(h, wse2b_ref[...], preferred_element_type=f32)
                   + 3.0, 0.0, 6.0) * (1.0 / 6.0)

    zc = jnp.zeros((H, 1, exp), f32)
    acc1 = (wd1[0:1] * jnp.concatenate([zc, y3[:, :W - 1, :]], axis=1)
            + wd1[1:2] * y3
            + wd1[2:3] * jnp.concatenate([y3[:, 1:, :], zc], axis=1) + bd1)
    x1 = _hswish(acc1 * se1).reshape(Mo, exp).astype(jnp.bfloat16)
    zr = jnp.zeros((1, W, exp), f32)
    acc2 = (wd2[0:1] * jnp.concatenate([zr, y3[:H - 1]], axis=0)
            + wd2[1:2] * y3
            + wd2[2:3] * jnp.concatenate([y3[1:], zr], axis=0) + bd2)
    x2 = _hswish(acc2 * se2).reshape(Mo, exp).astype(jnp.bfloat16)

    dn = (((1,), (1,)), ((), ()))
    out = lax.dot_general(w2a_ref[...], x1, dn, preferred_element_type=f32)
    out = out + lax.dot_general(w2b_ref[...], x2, dn, preferred_element_type=f32)
    o_ref[0] = (out + b2_ref[...]).reshape(oup, H, W)


def kernel(x_nchw, w1, bn1_s, bn1_b, wd1, bnd1_s, bnd1_b, wd2, bnd2_s, bnd2_b,
           w_se1, w_se2, w2, bn2_s, bn2_b):
    f32, bf16 = jnp.float32, jnp.bfloat16
    B, inC, H, W = x_nchw.shape
    exp = w1.shape[1]
    oup = w2.shape[1]

    w1f = (w1 * bn1_s).astype(bf16)
    b1 = bn1_b.astype(f32)
    wd1f = (wd1 * bnd1_s).astype(f32)
    wd2f = (wd2 * bnd2_s).astype(f32)
    bd1 = bnd1_b.astype(f32)
    bd2 = bnd2_b.astype(f32)
    w2f = w2 * bn2_s
    w2a = jnp.transpose(w2f[:exp]).astype(bf16)
    w2b = jnp.transpose(w2f[exp:]).astype(bf16)
    b2 = bn2_b.reshape(oup, 1).astype(f32)
    wse1a = w_se1[:exp].astype(f32)
    wse1b = w_se1[exp:].astype(f32)
    wse2a = w_se2[:, :exp].astype(f32)
    wse2b = w_se2[:, exp:].astype(f32)

    const = lambda shape: pl.BlockSpec(shape, lambda b: tuple(0 for _ in shape))
    out = pl.pallas_call(
        _block_kernel,
        out_shape=jax.ShapeDtypeStruct((B, oup, H, W), f32),
        grid=(B,),
        in_specs=[
            pl.BlockSpec((1, inC, H, W), lambda b: (b, 0, 0, 0)),
            const(w1f.shape), const(b1.shape),
            const(wd1f.shape), const(bd1.shape),
            const(wd2f.shape), const(bd2.shape),
            const(wse1a.shape), const(wse1b.shape),
            const(wse2a.shape), const(wse2b.shape),
            const(w2a.shape), const(w2b.shape), const(b2.shape),
        ],
        out_specs=pl.BlockSpec((1, oup, H, W), lambda b: (b, 0, 0, 0)),
        compiler_params=pltpu.CompilerParams(
            dimension_semantics=("parallel",),
            vmem_limit_bytes=64 * 1024 * 1024),
    )(x_nchw, w1f, b1, wd1f, bd1, wd2f, bd2,
      wse1a, wse1b, wse2a, wse2b, w2a, w2b, b2)
    return out


# speedup vs baseline: 1.7462x; 1.3815x over previous
import jax
import jax.numpy as jnp
from jax import lax
from jax.experimental import pallas as pl
from jax.experimental.pallas import tpu as pltpu

_PADL = 8


def _block_kernel(x_ref, w1_ref, b1_ref, wd1_ref, bd1_ref, wd2_ref, bd2_ref,
                  wse1a_ref, wse1b_ref, wse2a_ref, wse2b_ref,
                  w2a_ref, w2b_ref, b2_ref, o_ref, s_ref):
    _, inC, Mo = x_ref.shape
    exp = w1_ref.shape[1]
    H = s_ref.shape[0] - 2
    W = Mo // H
    f32 = jnp.float32

    y = lax.dot_general(x_ref[0], w1_ref[...], (((0,), (0,)), ((), ())),
                        preferred_element_type=f32)
    y = y + b1_ref[...]
    y = y * jnp.clip(y + 3.0, 0.0, 6.0)
    y3 = y.reshape(H, W, exp)

    z = jnp.zeros((1, 1, exp), f32)
    s_ref[:, _PADL - 1:_PADL, :] = jnp.broadcast_to(z, (H + 2, 1, exp))
    s_ref[:, _PADL + W:_PADL + W + 1, :] = jnp.broadcast_to(z, (H + 2, 1, exp))
    s_ref[0:1, _PADL:_PADL + W, :] = jnp.broadcast_to(z, (1, W, exp))
    s_ref[H + 1:H + 2, _PADL:_PADL + W, :] = jnp.broadcast_to(z, (1, W, exp))
    s_ref[1:H + 1, _PADL:_PADL + W, :] = y3

    wd1 = wd1_ref[...]
    wd2 = wd2_ref[...]
    bd1 = bd1_ref[...]
    bd2 = bd2_ref[...]

    S = jnp.sum(y, axis=0, keepdims=True)
    cs0 = jnp.sum(y3[:, 0, :], axis=0, keepdims=True)
    csW = jnp.sum(y3[:, W - 1, :], axis=0, keepdims=True)
    rs0 = jnp.sum(y3[0], axis=0, keepdims=True)
    rsH = jnp.sum(y3[H - 1], axis=0, keepdims=True)
    inv = 1.0 / float(Mo)
    p1 = (wd1[0:1] * (S - csW) + wd1[1:2] * S + wd1[2:3] * (S - cs0)) * inv + bd1
    p2 = (wd2[0:1] * (S - rsH) + wd2[1:2] * S + wd2[2:3] * (S - rs0)) * inv + bd2

    h = (jnp.dot(p1, wse1a_ref[...], preferred_element_type=f32)
         + jnp.dot(p2, wse1b_ref[...], preferred_element_type=f32))
    h = jnp.maximum(h, 0.0)
    se1 = jnp.clip(jnp.dot(h, wse2a_ref[...], preferred_element_type=f32)
                   + 3.0, 0.0, 6.0) * (1.0 / 6.0)
    se2 = jnp.clip(jnp.dot(h, wse2b_ref[...], preferred_element_type=f32)
                   + 3.0, 0.0, 6.0) * (1.0 / 6.0)
    wd1s = wd1 * se1
    wd2s = wd2 * se2
    bd1s = bd1 * se1
    bd2s = bd2 * se2

    u1 = (wd1s[0:1] * s_ref[1:H + 1, _PADL - 1:_PADL - 1 + W, :]
          + wd1s[1:2] * s_ref[1:H + 1, _PADL:_PADL + W, :]
          + wd1s[2:3] * s_ref[1:H + 1, _PADL + 1:_PADL + 1 + W, :] + bd1s)
    x1 = (u1 * jnp.clip(u1 + 3.0, 0.0, 6.0)).reshape(Mo, exp).astype(jnp.bfloat16)
    u2 = (wd2s[0:1] * s_ref[0:H, _PADL:_PADL + W, :]
          + wd2s[1:2] * s_ref[1:H + 1, _PADL:_PADL + W, :]
          + wd2s[2:3] * s_ref[2:H + 2, _PADL:_PADL + W, :] + bd2s)
    x2 = (u2 * jnp.clip(u2 + 3.0, 0.0, 6.0)).reshape(Mo, exp).astype(jnp.bfloat16)

    dn = (((1,), (1,)), ((), ()))
    out = lax.dot_general(w2a_ref[...], x1, dn, preferred_element_type=f32)
    out = out + lax.dot_general(w2b_ref[...], x2, dn, preferred_element_type=f32)
    o_ref[0] = out + b2_ref[...]


def kernel(x_nchw, w1, bn1_s, bn1_b, wd1, bnd1_s, bnd1_b, wd2, bnd2_s, bnd2_b,
           w_se1, w_se2, w2, bn2_s, bn2_b):
    f32, bf16 = jnp.float32, jnp.bfloat16
    B, inC, H, W = x_nchw.shape
    Mo = H * W
    exp = w1.shape[1]
    oup = w2.shape[1]

    w1f = (w1 * bn1_s).astype(bf16)
    b1 = bn1_b.astype(f32)
    wd1f = (wd1 * bnd1_s * (1.0 / 6.0)).astype(f32)
    wd2f = (wd2 * bnd2_s * (1.0 / 6.0)).astype(f32)
    bd1 = bnd1_b.astype(f32)
    bd2 = bnd2_b.astype(f32)
    w2f = w2 * bn2_s
    w2a = (jnp.transpose(w2f[:exp]) * (1.0 / 6.0)).astype(bf16)
    w2b = (jnp.transpose(w2f[exp:]) * (1.0 / 6.0)).astype(bf16)
    b2 = bn2_b.reshape(oup, 1).astype(f32)
    wse1a = w_se1[:exp].astype(f32)
    wse1b = w_se1[exp:].astype(f32)
    wse2a = w_se2[:, :exp].astype(f32)
    wse2b = w_se2[:, exp:].astype(f32)

    x3 = x_nchw.reshape(B, inC, Mo).astype(bf16)

    const = lambda shape: pl.BlockSpec(shape, lambda b: tuple(0 for _ in shape))
    out = pl.pallas_call(
        _block_kernel,
        out_shape=jax.ShapeDtypeStruct((B, oup, Mo), f32),
        grid=(B,),
        in_specs=[
            pl.BlockSpec((1, inC, Mo), lambda b: (b, 0, 0)),
            const(w1f.shape), const(b1.shape),
            const(wd1f.shape), const(bd1.shape),
            const(wd2f.shape), const(bd2.shape),
            const(wse1a.shape), const(wse1b.shape),
            const(wse2a.shape), const(wse2b.shape),
            const(w2a.shape), const(w2b.shape), const(b2.shape),
        ],
        out_specs=pl.BlockSpec((1, oup, Mo), lambda b: (b, 0, 0)),
        scratch_shapes=[pltpu.VMEM((H + 2, _PADL + W + 1, exp), f32)],
        compiler_params=pltpu.CompilerParams(
            dimension_semantics=("parallel",),
            vmem_limit_bytes=64 * 1024 * 1024),
    )(x3, w1f, b1, wd1f, bd1, wd2f, bd2,
      wse1a, wse1b, wse2a, wse2b, w2a, w2b, b2)
    return out.reshape(B, oup, H, W)


# speedup vs baseline: 1.8282x; 1.0469x over previous
import jax
import jax.numpy as jnp
from jax import lax
from jax.experimental import pallas as pl
from jax.experimental.pallas import tpu as pltpu


def _block_kernel(x_ref, w1_ref, b1_ref, wd1_ref, bd1_ref, wd2_ref,
                  bd2_ref, wse1a_ref, wse1b_ref, wse2a_ref, wse2b_ref,
                  w2a_ref, w2b_ref, b2_ref, o_ref, s_ref):
    _, inC, Mo = x_ref.shape
    exp = w1_ref.shape[1]
    H = s_ref.shape[0] - 2
    W = Mo // H
    f32 = jnp.float32

    y = lax.dot_general(x_ref[0], w1_ref[...], (((0,), (0,)), ((), ())),
                        preferred_element_type=f32)
    y = y + b1_ref[...]
    y = y * jnp.clip(y + 3.0, 0.0, 6.0)
    y3 = y.reshape(H, W, exp)

    z = jnp.zeros((1, W, exp), f32)
    s_ref[0:1] = z
    s_ref[H + 1:H + 2] = z
    s_ref[1:H + 1] = y3

    wd1 = wd1_ref[...]
    wd2 = wd2_ref[...]
    bd1 = bd1_ref[...]
    bd2 = bd2_ref[...]

    S = jnp.sum(y, axis=0, keepdims=True)
    cs0 = jnp.sum(y3[:, 0, :], axis=0, keepdims=True)
    csW = jnp.sum(y3[:, W - 1, :], axis=0, keepdims=True)
    rs0 = jnp.sum(y3[0], axis=0, keepdims=True)
    rsH = jnp.sum(y3[H - 1], axis=0, keepdims=True)
    inv = 1.0 / float(Mo)
    p1 = (wd1[0:1] * (S - csW) + wd1[1:2] * S + wd1[2:3] * (S - cs0)) * inv + bd1
    p2 = (wd2[0:1] * (S - rsH) + wd2[1:2] * S + wd2[2:3] * (S - rs0)) * inv + bd2

    h = (jnp.dot(p1, wse1a_ref[...], preferred_element_type=f32)
         + jnp.dot(p2, wse1b_ref[...], preferred_element_type=f32))
    h = jnp.maximum(h, 0.0)
    se1 = jnp.clip(jnp.dot(h, wse2a_ref[...], preferred_element_type=f32)
                   + 3.0, 0.0, 6.0) * (1.0 / 6.0)
    se2 = jnp.clip(jnp.dot(h, wse2b_ref[...], preferred_element_type=f32)
                   + 3.0, 0.0, 6.0) * (1.0 / 6.0)
    wd1s = wd1 * se1
    wd2s = wd2 * se2
    bd1s = (bd1 * se1).reshape(1, 1, exp)
    bd2s = (bd2 * se2).reshape(1, 1, exp)

    wi = lax.broadcasted_iota(jnp.int32, (1, W, 1), 1)
    W0 = jnp.where(wi == 0, 0.0, wd1s[0].reshape(1, 1, exp))
    W2 = jnp.where(wi == W - 1, 0.0, wd1s[2].reshape(1, 1, exp))
    u1 = (pltpu.roll(y3, 1, 1) * W0
          + y3 * wd1s[1].reshape(1, 1, exp)
          + pltpu.roll(y3, W - 1, 1) * W2 + bd1s)
    x1 = (u1 * jnp.clip(u1 + 3.0, 0.0, 6.0)).reshape(Mo, exp).astype(jnp.bfloat16)

    u2 = (wd2s[0].reshape(1, 1, exp) * s_ref[0:H]
          + wd2s[1].reshape(1, 1, exp) * s_ref[1:H + 1]
          + wd2s[2].reshape(1, 1, exp) * s_ref[2:H + 2] + bd2s)
    x2 = (u2 * jnp.clip(u2 + 3.0, 0.0, 6.0)).reshape(Mo, exp).astype(jnp.bfloat16)

    dn = (((1,), (1,)), ((), ()))
    out = lax.dot_general(w2a_ref[...], x1, dn, preferred_element_type=f32)
    out = out + lax.dot_general(w2b_ref[...], x2, dn, preferred_element_type=f32)
    o_ref[0] = out + b2_ref[...]


def kernel(x_nchw, w1, bn1_s, bn1_b, wd1, bnd1_s, bnd1_b, wd2, bnd2_s, bnd2_b,
           w_se1, w_se2, w2, bn2_s, bn2_b):
    f32, bf16 = jnp.float32, jnp.bfloat16
    B, inC, H, W = x_nchw.shape
    Mo = H * W
    exp = w1.shape[1]
    oup = w2.shape[1]

    w1f = (w1 * bn1_s).astype(bf16)
    b1 = bn1_b.astype(f32)
    wd1f = (wd1 * bnd1_s * (1.0 / 6.0)).astype(f32)
    wd2f = (wd2 * bnd2_s * (1.0 / 6.0)).astype(f32)
    bd1 = bnd1_b.astype(f32)
    bd2 = bnd2_b.astype(f32)
    w2f = w2 * bn2_s
    w2a = (jnp.transpose(w2f[:exp]) * (1.0 / 6.0)).astype(bf16)
    w2b = (jnp.transpose(w2f[exp:]) * (1.0 / 6.0)).astype(bf16)
    b2 = bn2_b.reshape(oup, 1).astype(f32)
    wse1a = w_se1[:exp].astype(f32)
    wse1b = w_se1[exp:].astype(f32)
    wse2a = w_se2[:, :exp].astype(f32)
    wse2b = w_se2[:, exp:].astype(f32)

    x3 = x_nchw.reshape(B, inC, Mo).astype(bf16)

    const = lambda shape: pl.BlockSpec(shape, lambda b: tuple(0 for _ in shape))
    out = pl.pallas_call(
        _block_kernel,
        out_shape=jax.ShapeDtypeStruct((B, oup, Mo), f32),
        grid=(B,),
        in_specs=[
            pl.BlockSpec((1, inC, Mo), lambda b: (b, 0, 0)),
            const(w1f.shape), const(b1.shape),
            const(wd1f.shape), const(bd1.shape),
            const(wd2f.shape), const(bd2.shape),
            const(wse1a.shape), const(wse1b.shape),
            const(wse2a.shape), const(wse2b.shape),
            const(w2a.shape), const(w2b.shape), const(b2.shape),
        ],
        out_specs=pl.BlockSpec((1, oup, Mo), lambda b: (b, 0, 0)),
        scratch_shapes=[pltpu.VMEM((H + 2, W, exp), f32)],
        compiler_params=pltpu.CompilerParams(
            dimension_semantics=("arbitrary",),
            vmem_limit_bytes=64 * 1024 * 1024),
    )(x3, w1f, b1, wd1f, bd1, wd2f, bd2,
      wse1a, wse1b, wse2a, wse2b, w2a, w2b, b2)
    return out.reshape(B, oup, H, W)


# speedup vs baseline: 1.8780x; 1.0273x over previous
import jax
import jax.numpy as jnp
from jax import lax
from jax.experimental import pallas as pl
from jax.experimental.pallas import tpu as pltpu


def _block_kernel(x_ref, w1_ref, b1_ref, wd1_ref, bd1_ref, wd2_ref,
                  bd2_ref, wse1a_ref, wse1b_ref, wse2a_ref, wse2b_ref,
                  w2a_ref, w2b_ref, b2_ref, o_ref, s_ref):
    _, inC, Mo = x_ref.shape
    exp = w1_ref.shape[1]
    H = s_ref.shape[0] - 2
    W = Mo // H
    f32 = jnp.float32

    y = lax.dot_general(x_ref[0], w1_ref[...], (((0,), (0,)), ((), ())),
                        preferred_element_type=f32)
    y = y + b1_ref[...]
    y = y * jnp.clip(y + 3.0, 0.0, 6.0)
    y3 = y.reshape(H, W, exp)

    z = jnp.zeros((1, W, exp), f32)
    s_ref[0:1] = z
    s_ref[H + 1:H + 2] = z
    s_ref[1:H + 1] = y3

    wd1 = wd1_ref[...]
    wd2 = wd2_ref[...]
    bd1 = bd1_ref[...]
    bd2 = bd2_ref[...]

    S = jnp.sum(y, axis=0, keepdims=True)
    cs0 = jnp.sum(y3[:, 0, :], axis=0, keepdims=True)
    csW = jnp.sum(y3[:, W - 1, :], axis=0, keepdims=True)
    rs0 = jnp.sum(y3[0], axis=0, keepdims=True)
    rsH = jnp.sum(y3[H - 1], axis=0, keepdims=True)
    inv = 1.0 / float(Mo)
    p1 = (wd1[0:1] * (S - csW) + wd1[1:2] * S + wd1[2:3] * (S - cs0)) * inv + bd1
    p2 = (wd2[0:1] * (S - rsH) + wd2[1:2] * S + wd2[2:3] * (S - rs0)) * inv + bd2

    h = (jnp.dot(p1, wse1a_ref[...], preferred_element_type=f32)
         + jnp.dot(p2, wse1b_ref[...], preferred_element_type=f32))
    h = jnp.maximum(h, 0.0)
    se1 = jnp.clip(jnp.dot(h, wse2a_ref[...], preferred_element_type=f32)
                   + 3.0, 0.0, 6.0) * (1.0 / 6.0)
    se2 = jnp.clip(jnp.dot(h, wse2b_ref[...], preferred_element_type=f32)
                   + 3.0, 0.0, 6.0) * (1.0 / 6.0)
    wd1s = wd1 * se1
    wd2s = wd2 * se2
    bd1s = (bd1 * se1).reshape(1, 1, exp)
    bd2s = (bd2 * se2).reshape(1, 1, exp)

    wi = lax.broadcasted_iota(jnp.int32, (1, W, 1), 1)
    W0 = jnp.where(wi == 0, 0.0, wd1s[0].reshape(1, 1, exp))
    W2 = jnp.where(wi == W - 1, 0.0, wd1s[2].reshape(1, 1, exp))
    u1 = (pltpu.roll(y3, 1, 1) * W0
          + y3 * wd1s[1].reshape(1, 1, exp)
          + pltpu.roll(y3, W - 1, 1) * W2 + bd1s)
    x1 = (u1 * jnp.clip(u1 + 3.0, 0.0, 6.0)).reshape(Mo, exp).astype(jnp.bfloat16)

    u2 = (wd2s[0].reshape(1, 1, exp) * s_ref[0:H]
          + wd2s[1].reshape(1, 1, exp) * s_ref[1:H + 1]
          + wd2s[2].reshape(1, 1, exp) * s_ref[2:H + 2] + bd2s)
    x2 = (u2 * jnp.clip(u2 + 3.0, 0.0, 6.0)).reshape(Mo, exp).astype(jnp.bfloat16)

    dn = (((1,), (1,)), ((), ()))
    out = lax.dot_general(w2a_ref[...], x1, dn, preferred_element_type=f32)
    out = out + lax.dot_general(w2b_ref[...], x2, dn, preferred_element_type=f32)
    o_ref[0] = (out + b2_ref[...]).astype(jnp.bfloat16)


def kernel(x_nchw, w1, bn1_s, bn1_b, wd1, bnd1_s, bnd1_b, wd2, bnd2_s, bnd2_b,
           w_se1, w_se2, w2, bn2_s, bn2_b):
    f32, bf16 = jnp.float32, jnp.bfloat16
    B, inC, H, W = x_nchw.shape
    Mo = H * W
    exp = w1.shape[1]
    oup = w2.shape[1]

    w1f = (w1 * bn1_s).astype(bf16)
    b1 = bn1_b.astype(f32)
    wd1f = (wd1 * bnd1_s * (1.0 / 6.0)).astype(f32)
    wd2f = (wd2 * bnd2_s * (1.0 / 6.0)).astype(f32)
    bd1 = bnd1_b.astype(f32)
    bd2 = bnd2_b.astype(f32)
    w2f = w2 * bn2_s
    w2a = (jnp.transpose(w2f[:exp]) * (1.0 / 6.0)).astype(bf16)
    w2b = (jnp.transpose(w2f[exp:]) * (1.0 / 6.0)).astype(bf16)
    b2 = bn2_b.reshape(oup, 1).astype(f32)
    wse1a = w_se1[:exp].astype(f32)
    wse1b = w_se1[exp:].astype(f32)
    wse2a = w_se2[:, :exp].astype(f32)
    wse2b = w_se2[:, exp:].astype(f32)

    x3 = x_nchw.reshape(B, inC, Mo).astype(bf16)

    const = lambda shape: pl.BlockSpec(shape, lambda b: tuple(0 for _ in shape))
    out = pl.pallas_call(
        _block_kernel,
        out_shape=jax.ShapeDtypeStruct((B, oup, Mo), bf16),
        grid=(B,),
        in_specs=[
            pl.BlockSpec((1, inC, Mo), lambda b: (b, 0, 0)),
            const(w1f.shape), const(b1.shape),
            const(wd1f.shape), const(bd1.shape),
            const(wd2f.shape), const(bd2.shape),
            const(wse1a.shape), const(wse1b.shape),
            const(wse2a.shape), const(wse2b.shape),
            const(w2a.shape), const(w2b.shape), const(b2.shape),
        ],
        out_specs=pl.BlockSpec((1, oup, Mo), lambda b: (b, 0, 0)),
        scratch_shapes=[pltpu.VMEM((H + 2, W, exp), f32)],
        compiler_params=pltpu.CompilerParams(
            dimension_semantics=("arbitrary",),
            vmem_limit_bytes=64 * 1024 * 1024),
    )(x3, w1f, b1, wd1f, bd1, wd2f, bd2,
      wse1a, wse1b, wse2a, wse2b, w2a, w2b, b2)
    return out.reshape(B, oup, H, W).astype(f32)


# speedup vs baseline: 1.9891x; 1.0591x over previous
import functools

import jax
import jax.numpy as jnp
from jax import lax
from jax.experimental import pallas as pl
from jax.experimental.pallas import tpu as pltpu


def _block_kernel(x_ref, w1_ref, b1_ref, wd1_ref, bd1_ref, wd2_ref,
                  bd2_ref, wse1a_ref, wse1b_ref, wse2a_ref, wse2b_ref,
                  w2a_ref, w2b_ref, b2_ref, o_ref, *, H):
    _, inC, Mo = x_ref.shape
    exp = w1_ref.shape[1]
    W = Mo // H
    f32 = jnp.float32

    y = lax.dot_general(x_ref[0].astype(jnp.bfloat16), w1_ref[...],
                        (((0,), (0,)), ((), ())),
                        preferred_element_type=f32)
    y = y + b1_ref[...]
    y = y * jnp.clip(y + 3.0, 0.0, 6.0)
    y3 = y.reshape(H, W, exp)

    wd1 = wd1_ref[...]
    wd2 = wd2_ref[...]
    bd1 = bd1_ref[...]
    bd2 = bd2_ref[...]

    S = jnp.sum(y, axis=0, keepdims=True)
    cs0 = jnp.sum(y3[:, 0, :], axis=0, keepdims=True)
    csW = jnp.sum(y3[:, W - 1, :], axis=0, keepdims=True)
    rs0 = jnp.sum(y3[0], axis=0, keepdims=True)
    rsH = jnp.sum(y3[H - 1], axis=0, keepdims=True)
    inv = 1.0 / float(Mo)
    p1 = (wd1[0:1] * (S - csW) + wd1[1:2] * S + wd1[2:3] * (S - cs0)) * inv + bd1
    p2 = (wd2[0:1] * (S - rsH) + wd2[1:2] * S + wd2[2:3] * (S - rs0)) * inv + bd2

    h = (jnp.dot(p1, wse1a_ref[...], preferred_element_type=f32)
         + jnp.dot(p2, wse1b_ref[...], preferred_element_type=f32))
    h = jnp.maximum(h, 0.0)
    se1 = jnp.clip(jnp.dot(h, wse2a_ref[...], preferred_element_type=f32)
                   + 3.0, 0.0, 6.0) * (1.0 / 6.0)
    se2 = jnp.clip(jnp.dot(h, wse2b_ref[...], preferred_element_type=f32)
                   + 3.0, 0.0, 6.0) * (1.0 / 6.0)
    wd1s = wd1 * se1
    wd2s = wd2 * se2
    bd1s = (bd1 * se1).reshape(1, 1, exp)
    bd2s = (bd2 * se2).reshape(1, 1, exp)

    wi = lax.broadcasted_iota(jnp.int32, (1, W, 1), 1)
    W0 = jnp.where(wi == 0, 0.0, wd1s[0].reshape(1, 1, exp))
    W2 = jnp.where(wi == W - 1, 0.0, wd1s[2].reshape(1, 1, exp))
    u1 = (pltpu.roll(y3, 1, 1) * W0
          + y3 * wd1s[1].reshape(1, 1, exp)
          + pltpu.roll(y3, W - 1, 1) * W2 + bd1s)
    x1 = (u1 * jnp.clip(u1 + 3.0, 0.0, 6.0)).reshape(Mo, exp).astype(jnp.bfloat16)

    hi = lax.broadcasted_iota(jnp.int32, (H, 1, 1), 0)
    V0 = jnp.where(hi == 0, 0.0, wd2s[0].reshape(1, 1, exp))
    V2 = jnp.where(hi == H - 1, 0.0, wd2s[2].reshape(1, 1, exp))
    u2 = (pltpu.roll(y3, 1, 0) * V0
          + y3 * wd2s[1].reshape(1, 1, exp)
          + pltpu.roll(y3, H - 1, 0) * V2 + bd2s)
    x2 = (u2 * jnp.clip(u2 + 3.0, 0.0, 6.0)).reshape(Mo, exp).astype(jnp.bfloat16)

    dn = (((1,), (1,)), ((), ()))
    out = lax.dot_general(w2a_ref[...], x1, dn, preferred_element_type=f32)
    out = out + lax.dot_general(w2b_ref[...], x2, dn, preferred_element_type=f32)
    o_ref[0] = (out + b2_ref[...]).astype(jnp.bfloat16)


def kernel(x_nchw, w1, bn1_s, bn1_b, wd1, bnd1_s, bnd1_b, wd2, bnd2_s, bnd2_b,
           w_se1, w_se2, w2, bn2_s, bn2_b):
    f32, bf16 = jnp.float32, jnp.bfloat16
    B, inC, H, W = x_nchw.shape
    Mo = H * W
    exp = w1.shape[1]
    oup = w2.shape[1]

    w1f = (w1 * bn1_s).astype(bf16)
    b1 = bn1_b.astype(f32)
    wd1f = (wd1 * bnd1_s * (1.0 / 6.0)).astype(f32)
    wd2f = (wd2 * bnd2_s * (1.0 / 6.0)).astype(f32)
    bd1 = bnd1_b.astype(f32)
    bd2 = bnd2_b.astype(f32)
    w2f = w2 * bn2_s
    w2a = (jnp.transpose(w2f[:exp]) * (1.0 / 6.0)).astype(bf16)
    w2b = (jnp.transpose(w2f[exp:]) * (1.0 / 6.0)).astype(bf16)
    b2 = bn2_b.reshape(oup, 1).astype(f32)
    wse1a = w_se1[:exp].astype(f32)
    wse1b = w_se1[exp:].astype(f32)
    wse2a = w_se2[:, :exp].astype(f32)
    wse2b = w_se2[:, exp:].astype(f32)

    x3 = x_nchw.reshape(B, inC, Mo)

    const = lambda shape: pl.BlockSpec(shape, lambda b: tuple(0 for _ in shape))
    out = pl.pallas_call(
        functools.partial(_block_kernel, H=H),
        out_shape=jax.ShapeDtypeStruct((B, oup, Mo), bf16),
        grid=(B,),
        in_specs=[
            pl.BlockSpec((1, inC, Mo), lambda b: (b, 0, 0)),
            const(w1f.shape), const(b1.shape),
            const(wd1f.shape), const(bd1.shape),
            const(wd2f.shape), const(bd2.shape),
            const(wse1a.shape), const(wse1b.shape),
            const(wse2a.shape), const(wse2b.shape),
            const(w2a.shape), const(w2b.shape), const(b2.shape),
        ],
        out_specs=pl.BlockSpec((1, oup, Mo), lambda b: (b, 0, 0)),
        compiler_params=pltpu.CompilerParams(
            dimension_semantics=("arbitrary",),
            vmem_limit_bytes=64 * 1024 * 1024),
    )(x3, w1f, b1, wd1f, bd1, wd2f, bd2,
      wse1a, wse1b, wse2a, wse2b, w2a, w2b, b2)
    return out.reshape(B, oup, H, W).astype(f32)


# speedup vs baseline: 2.0139x; 1.0125x over previous
import functools

import jax
import jax.numpy as jnp
from jax import lax
from jax.experimental import pallas as pl
from jax.experimental.pallas import tpu as pltpu


def _block_kernel(x_ref, w1_ref, b1_ref, wd1_ref, bd1_ref, wd2_ref,
                  bd2_ref, wse1a_ref, wse1b_ref, wse2a_ref, wse2b_ref,
                  w2a_ref, w2b_ref, b2_ref, o_ref, *, H):
    nb, inC, Mo = x_ref.shape
    exp = w1_ref.shape[1]
    W = Mo // H
    f32 = jnp.float32

    for j in range(nb):
        _one_batch(x_ref, w1_ref, b1_ref, wd1_ref, bd1_ref, wd2_ref,
                   bd2_ref, wse1a_ref, wse1b_ref, wse2a_ref, wse2b_ref,
                   w2a_ref, w2b_ref, b2_ref, o_ref, j, H, W, Mo, exp, f32)


def _one_batch(x_ref, w1_ref, b1_ref, wd1_ref, bd1_ref, wd2_ref,
               bd2_ref, wse1a_ref, wse1b_ref, wse2a_ref, wse2b_ref,
               w2a_ref, w2b_ref, b2_ref, o_ref, j, H, W, Mo, exp, f32):
    y = lax.dot_general(x_ref[j].astype(jnp.bfloat16), w1_ref[...],
                        (((0,), (0,)), ((), ())),
                        preferred_element_type=f32)
    y = y + b1_ref[...]
    y = y * jnp.clip(y + 3.0, 0.0, 6.0)
    y3 = y.reshape(H, W, exp)

    wd1 = wd1_ref[...]
    wd2 = wd2_ref[...]
    bd1 = bd1_ref[...]
    bd2 = bd2_ref[...]

    S = jnp.sum(y, axis=0, keepdims=True)
    cs0 = jnp.sum(y3[:, 0, :], axis=0, keepdims=True)
    csW = jnp.sum(y3[:, W - 1, :], axis=0, keepdims=True)
    rs0 = jnp.sum(y3[0], axis=0, keepdims=True)
    rsH = jnp.sum(y3[H - 1], axis=0, keepdims=True)
    inv = 1.0 / float(Mo)
    p1 = (wd1[0:1] * (S - csW) + wd1[1:2] * S + wd1[2:3] * (S - cs0)) * inv + bd1
    p2 = (wd2[0:1] * (S - rsH) + wd2[1:2] * S + wd2[2:3] * (S - rs0)) * inv + bd2

    h = (jnp.dot(p1, wse1a_ref[...], preferred_element_type=f32)
         + jnp.dot(p2, wse1b_ref[...], preferred_element_type=f32))
    h = jnp.maximum(h, 0.0)
    se1 = jnp.clip(jnp.dot(h, wse2a_ref[...], preferred_element_type=f32)
                   + 3.0, 0.0, 6.0) * (1.0 / 6.0)
    se2 = jnp.clip(jnp.dot(h, wse2b_ref[...], preferred_element_type=f32)
                   + 3.0, 0.0, 6.0) * (1.0 / 6.0)
    wd1s = wd1 * se1
    wd2s = wd2 * se2
    bd1s = (bd1 * se1).reshape(1, 1, exp)
    bd2s = (bd2 * se2).reshape(1, 1, exp)

    wi = lax.broadcasted_iota(jnp.int32, (1, W, 1), 1)
    W0 = jnp.where(wi == 0, 0.0, wd1s[0].reshape(1, 1, exp))
    W2 = jnp.where(wi == W - 1, 0.0, wd1s[2].reshape(1, 1, exp))
    u1 = (pltpu.roll(y3, 1, 1) * W0
          + y3 * wd1s[1].reshape(1, 1, exp)
          + pltpu.roll(y3, W - 1, 1) * W2 + bd1s)
    x1 = (u1 * jnp.clip(u1 + 3.0, 0.0, 6.0)).reshape(Mo, exp).astype(jnp.bfloat16)

    hi = lax.broadcasted_iota(jnp.int32, (H, 1, 1), 0)
    V0 = jnp.where(hi == 0, 0.0, wd2s[0].reshape(1, 1, exp))
    V2 = jnp.where(hi == H - 1, 0.0, wd2s[2].reshape(1, 1, exp))
    u2 = (pltpu.roll(y3, 1, 0) * V0
          + y3 * wd2s[1].reshape(1, 1, exp)
          + pltpu.roll(y3, H - 1, 0) * V2 + bd2s)
    x2 = (u2 * jnp.clip(u2 + 3.0, 0.0, 6.0)).reshape(Mo, exp).astype(jnp.bfloat16)

    dn = (((1,), (1,)), ((), ()))
    out = lax.dot_general(w2a_ref[...], x1, dn, preferred_element_type=f32)
    out = out + lax.dot_general(w2b_ref[...], x2, dn, preferred_element_type=f32)
    o_ref[j] = (out + b2_ref[...]).astype(jnp.bfloat16)


def kernel(x_nchw, w1, bn1_s, bn1_b, wd1, bnd1_s, bnd1_b, wd2, bnd2_s, bnd2_b,
           w_se1, w_se2, w2, bn2_s, bn2_b):
    f32, bf16 = jnp.float32, jnp.bfloat16
    B, inC, H, W = x_nchw.shape
    Mo = H * W
    exp = w1.shape[1]
    oup = w2.shape[1]

    w1f = (w1 * bn1_s).astype(bf16)
    b1 = bn1_b.astype(f32)
    wd1f = (wd1 * bnd1_s * (1.0 / 6.0)).astype(f32)
    wd2f = (wd2 * bnd2_s * (1.0 / 6.0)).astype(f32)
    bd1 = bnd1_b.astype(f32)
    bd2 = bnd2_b.astype(f32)
    w2f = w2 * bn2_s
    w2a = (jnp.transpose(w2f[:exp]) * (1.0 / 6.0)).astype(bf16)
    w2b = (jnp.transpose(w2f[exp:]) * (1.0 / 6.0)).astype(bf16)
    b2 = bn2_b.reshape(oup, 1).astype(f32)
    wse1a = w_se1[:exp].astype(f32)
    wse1b = w_se1[exp:].astype(f32)
    wse2a = w_se2[:, :exp].astype(f32)
    wse2b = w_se2[:, exp:].astype(f32)

    x3 = x_nchw.reshape(B, inC, Mo)

    const = lambda shape: pl.BlockSpec(shape, lambda b: tuple(0 for _ in shape))
    NB = 2
    out = pl.pallas_call(
        functools.partial(_block_kernel, H=H),
        out_shape=jax.ShapeDtypeStruct((B, oup, Mo), bf16),
        grid=(B // NB,),
        in_specs=[
            pl.BlockSpec((NB, inC, Mo), lambda b: (b, 0, 0)),
            const(w1f.shape), const(b1.shape),
            const(wd1f.shape), const(bd1.shape),
            const(wd2f.shape), const(bd2.shape),
            const(wse1a.shape), const(wse1b.shape),
            const(wse2a.shape), const(wse2b.shape),
            const(w2a.shape), const(w2b.shape), const(b2.shape),
        ],
        out_specs=pl.BlockSpec((NB, oup, Mo), lambda b: (b, 0, 0)),
        compiler_params=pltpu.CompilerParams(
            dimension_semantics=("arbitrary",),
            vmem_limit_bytes=64 * 1024 * 1024),
    )(x3, w1f, b1, wd1f, bd1, wd2f, bd2,
      wse1a, wse1b, wse2a, wse2b, w2a, w2b, b2)
    return out.reshape(B, oup, H, W).astype(f32)


# speedup vs baseline: 2.0663x; 1.0260x over previous
import functools

import jax
import jax.numpy as jnp
from jax import lax
from jax.experimental import pallas as pl
from jax.experimental.pallas import tpu as pltpu


def _block_kernel(x_ref, w1_ref, b1_ref, wd1_ref, bd1_ref, wd2_ref,
                  bd2_ref, wse1a_ref, wse1b_ref, wse2a_ref, wse2b_ref,
                  w2a_ref, w2b_ref, b2_ref, o_ref, *, H):
    nb, inC, Mo = x_ref.shape
    exp = w1_ref.shape[1]
    W = Mo // H
    f32 = jnp.float32

    for j in range(nb):
        _one_batch(x_ref, w1_ref, b1_ref, wd1_ref, bd1_ref, wd2_ref,
                   bd2_ref, wse1a_ref, wse1b_ref, wse2a_ref, wse2b_ref,
                   w2a_ref, w2b_ref, b2_ref, o_ref, j, H, W, Mo, exp, f32)


def _one_batch(x_ref, w1_ref, b1_ref, wd1_ref, bd1_ref, wd2_ref,
               bd2_ref, wse1a_ref, wse1b_ref, wse2a_ref, wse2b_ref,
               w2a_ref, w2b_ref, b2_ref, o_ref, j, H, W, Mo, exp, f32):
    y = lax.dot_general(x_ref[j].astype(jnp.bfloat16), w1_ref[...],
                        (((0,), (0,)), ((), ())),
                        preferred_element_type=f32)
    y = y + b1_ref[...]
    y = y * jnp.clip(y + 3.0, 0.0, 6.0)
    y3 = y.reshape(H, W, exp)

    wd1 = wd1_ref[...]
    wd2 = wd2_ref[...]
    bd1 = bd1_ref[...]
    bd2 = bd2_ref[...]

    S = jnp.sum(y, axis=0, keepdims=True)
    cs0 = jnp.sum(y3[:, 0, :], axis=0, keepdims=True)
    csW = jnp.sum(y3[:, W - 1, :], axis=0, keepdims=True)
    rs0 = jnp.sum(y3[0], axis=0, keepdims=True)
    rsH = jnp.sum(y3[H - 1], axis=0, keepdims=True)
    inv = 1.0 / float(Mo)
    p1 = (wd1[0:1] * (S - csW) + wd1[1:2] * S + wd1[2:3] * (S - cs0)) * inv + bd1
    p2 = (wd2[0:1] * (S - rsH) + wd2[1:2] * S + wd2[2:3] * (S - rs0)) * inv + bd2

    h = (jnp.dot(p1, wse1a_ref[...], preferred_element_type=f32)
         + jnp.dot(p2, wse1b_ref[...], preferred_element_type=f32))
    h = jnp.maximum(h, 0.0)
    se1 = jnp.clip(jnp.dot(h, wse2a_ref[...], preferred_element_type=f32)
                   + 3.0, 0.0, 6.0) * (1.0 / 6.0)
    se2 = jnp.clip(jnp.dot(h, wse2b_ref[...], preferred_element_type=f32)
                   + 3.0, 0.0, 6.0) * (1.0 / 6.0)
    wd1s = wd1 * se1
    wd2s = wd2 * se2
    bd1s = (bd1 * se1).reshape(1, 1, exp)
    bd2s = (bd2 * se2).reshape(1, 1, exp)

    zc = jnp.zeros((H, 1, exp), f32)
    u1 = (wd1s[0].reshape(1, 1, exp) * jnp.concatenate([zc, y3[:, :W - 1]], 1)
          + wd1s[1].reshape(1, 1, exp) * y3
          + wd1s[2].reshape(1, 1, exp) * jnp.concatenate([y3[:, 1:], zc], 1)
          + bd1s)
    x1 = (u1 * jnp.clip(u1 + 3.0, 0.0, 6.0)).reshape(Mo, exp).astype(jnp.bfloat16)

    zr = jnp.zeros((1, W, exp), f32)
    u2 = (wd2s[0].reshape(1, 1, exp) * jnp.concatenate([zr, y3[:H - 1]], 0)
          + wd2s[1].reshape(1, 1, exp) * y3
          + wd2s[2].reshape(1, 1, exp) * jnp.concatenate([y3[1:], zr], 0)
          + bd2s)
    x2 = (u2 * jnp.clip(u2 + 3.0, 0.0, 6.0)).reshape(Mo, exp).astype(jnp.bfloat16)

    dn = (((1,), (1,)), ((), ()))
    out = lax.dot_general(w2a_ref[...], x1, dn, preferred_element_type=f32)
    out = out + lax.dot_general(w2b_ref[...], x2, dn, preferred_element_type=f32)
    o_ref[j] = (out + b2_ref[...]).astype(jnp.bfloat16)


def kernel(x_nchw, w1, bn1_s, bn1_b, wd1, bnd1_s, bnd1_b, wd2, bnd2_s, bnd2_b,
           w_se1, w_se2, w2, bn2_s, bn2_b):
    f32, bf16 = jnp.float32, jnp.bfloat16
    B, inC, H, W = x_nchw.shape
    Mo = H * W
    exp = w1.shape[1]
    oup = w2.shape[1]

    w1f = (w1 * bn1_s).astype(bf16)
    b1 = bn1_b.astype(f32)
    wd1f = (wd1 * bnd1_s * (1.0 / 6.0)).astype(f32)
    wd2f = (wd2 * bnd2_s * (1.0 / 6.0)).astype(f32)
    bd1 = bnd1_b.astype(f32)
    bd2 = bnd2_b.astype(f32)
    w2f = w2 * bn2_s
    w2a = (jnp.transpose(w2f[:exp]) * (1.0 / 6.0)).astype(bf16)
    w2b = (jnp.transpose(w2f[exp:]) * (1.0 / 6.0)).astype(bf16)
    b2 = bn2_b.reshape(oup, 1).astype(f32)
    wse1a = w_se1[:exp].astype(f32)
    wse1b = w_se1[exp:].astype(f32)
    wse2a = w_se2[:, :exp].astype(f32)
    wse2b = w_se2[:, exp:].astype(f32)

    x3 = x_nchw.reshape(B, inC, Mo)

    const = lambda shape: pl.BlockSpec(shape, lambda b: tuple(0 for _ in shape))
    NB = 2
    out = pl.pallas_call(
        functools.partial(_block_kernel, H=H),
        out_shape=jax.ShapeDtypeStruct((B, oup, Mo), bf16),
        grid=(B // NB,),
        in_specs=[
            pl.BlockSpec((NB, inC, Mo), lambda b: (b, 0, 0)),
            const(w1f.shape), const(b1.shape),
            const(wd1f.shape), const(bd1.shape),
            const(wd2f.shape), const(bd2.shape),
            const(wse1a.shape), const(wse1b.shape),
            const(wse2a.shape), const(wse2b.shape),
            const(w2a.shape), const(w2b.shape), const(b2.shape),
        ],
        out_specs=pl.BlockSpec((NB, oup, Mo), lambda b: (b, 0, 0)),
        compiler_params=pltpu.CompilerParams(
            dimension_semantics=("arbitrary",),
            vmem_limit_bytes=64 * 1024 * 1024),
    )(x3, w1f, b1, wd1f, bd1, wd2f, bd2,
      wse1a, wse1b, wse2a, wse2b, w2a, w2b, b2)
    return out.reshape(B, oup, H, W).astype(f32)
